# baseline (speedup 1.0000x reference)
import functools

import jax
import jax.numpy as jnp
from jax import lax
from jax.experimental import pallas as pl
from jax.experimental.pallas import tpu as pltpu
from jax.experimental.pallas import tpu_sc as plsc

_VOCAB = 100000
_EMB = 64
_RMAX = 7
_B = 1024
_L = 200
_NTOK = _B * _L

_WPAD = 128
_KPAD = 512

_NC, _NS = 2, 16
_NW = _NC * _NS
_NBUF = 2

_VB = 2048
_NBLK = (_VOCAB + _VB - 1) // _VB
_MAINW = (_VOCAB - (_NBLK - 1) * _VB) // 128 * 128
_VPAD = _NBLK * _VB

_NGRP = 4
_BG = _B // _NGRP
_TOKG = _BG * _L
_CHUNK = 80


def _tc_pack(KT, WT, ktail, wtail):
    def fetch(src_hbm, dst, sems, i, rows):
        b = lax.rem(i, 2)

        @pl.when(i < _NBLK - 1)
        def _():
            pltpu.make_async_copy(
                src_hbm.at[:, pl.ds(i * _VB, _VB)], dst.at[b], sems.at[b]
            ).start()

        @pl.when(i == _NBLK - 1)
        def _():
            pltpu.make_async_copy(
                src_hbm.at[:, pl.ds(i * _VB, _MAINW)],
                dst.at[b, :, pl.ds(0, _MAINW)], sems.at[b]
            ).start()

    def drain(src_hbm, dst, sems, i):
        b = lax.rem(i, 2)

        @pl.when(i < _NBLK - 1)
        def _():
            pltpu.make_async_copy(
                src_hbm.at[:, pl.ds(i * _VB, _VB)], dst.at[b], sems.at[b]
            ).wait()

        @pl.when(i == _NBLK - 1)
        def _():
            pltpu.make_async_copy(
                src_hbm.at[:, pl.ds(i * _VB, _MAINW)],
                dst.at[b, :, pl.ds(0, _MAINW)], sems.at[b]
            ).wait()

    def body(kt_hbm, wt_hbm, ktail_ref, wtail_ref, kp_ref, wp_ref,
             kt_v, wt_v, ksems, wsems):
        i = pl.program_id(0)
        b = lax.rem(i, 2)

        @pl.when(i == 0)
        def _():
            fetch(kt_hbm, kt_v, ksems, i, _VB)
            fetch(wt_hbm, wt_v, wsems, i, _VB)

        @pl.when(i < _NBLK - 1)
        def _():
            fetch(kt_hbm, kt_v, ksems, i + 1, _VB)
            fetch(wt_hbm, wt_v, wsems, i + 1, _VB)

        def kchunk(src_ref, lane0, s):
            cols = [src_ref[pl.ds(64 * j, 64), pl.ds(lane0, 128)].T
                    for j in range(_RMAX)]
            groups = [jnp.concatenate(cols[0:2], axis=1),
                      jnp.concatenate(cols[2:4], axis=1),
                      jnp.concatenate(cols[4:6], axis=1),
                      cols[6]]
            for g, val in enumerate(groups):
                kp_ref[pl.ds(128 * s, 128), pl.ds(128 * g, val.shape[1])] = val

        def wchunk(src_ref, lane0, s):
            wp_ref[pl.ds(128 * s, 128), pl.ds(0, 64)] = (
                src_ref[pl.ds(0, 64), pl.ds(lane0, 128)].T)

        drain(kt_hbm, kt_v, ksems, i)
        drain(wt_hbm, wt_v, wsems, i)

        @pl.when(i < _NBLK - 1)
        def _():
            for s in range(_VB // 128):
                kchunk(kt_v.at[b], 128 * s, s)
                wchunk(wt_v.at[b], 128 * s, s)

        @pl.when(i == _NBLK - 1)
        def _():
            for s in range(_MAINW // 128):
                kchunk(kt_v.at[b], 128 * s, s)
                wchunk(wt_v.at[b], 128 * s, s)
            kchunk(ktail_ref, 0, _MAINW // 128)
            wchunk(wtail_ref, 0, _MAINW // 128)

    return pl.pallas_call(
        body,
        grid=(_NBLK,),
        in_specs=[pl.BlockSpec(memory_space=pl.ANY),
                  pl.BlockSpec(memory_space=pl.ANY),
                  pl.BlockSpec((_RMAX * _EMB, 128), lambda i: (0, 0)),
                  pl.BlockSpec((_EMB, 128), lambda i: (0, 0))],
        out_specs=[pl.BlockSpec((_VB, _KPAD), lambda i: (i, 0)),
                   pl.BlockSpec((_VB, _WPAD), lambda i: (i, 0))],
        out_shape=[jax.ShapeDtypeStruct((_VPAD, _KPAD), jnp.float32),
                   jax.ShapeDtypeStruct((_VPAD, _WPAD), jnp.float32)],
        scratch_shapes=[
            pltpu.VMEM((2, _RMAX * _EMB, _VB), jnp.float32),
            pltpu.VMEM((2, _EMB, _VB), jnp.float32),
            pltpu.SemaphoreType.DMA((2,)),
            pltpu.SemaphoreType.DMA((2,)),
        ],
    )(KT, WT, ktail, wtail)


def _sc_gather(seq_grp, Wp, Kp):
    mesh = plsc.VectorSubcoreMesh(core_axis_name="c", subcore_axis_name="s")
    tok_per_w = _TOKG // _NW
    nround = tok_per_w // _CHUNK // _NBUF

    @functools.partial(
        pl.kernel,
        out_type=(
            jax.ShapeDtypeStruct((_TOKG, _WPAD), jnp.float32),
            jax.ShapeDtypeStruct((_TOKG, _KPAD), jnp.float32),
        ),
        mesh=mesh,
        scratch_types=[
            pltpu.VMEM((_NBUF, _CHUNK), jnp.int32),
            pltpu.VMEM((_NBUF, _CHUNK, _WPAD), jnp.float32),
            pltpu.VMEM((_NBUF, _CHUNK, _KPAD), jnp.float32),
            pltpu.SemaphoreType.DMA,
            pltpu.SemaphoreType.DMA,
            pltpu.SemaphoreType.DMA,
            pltpu.SemaphoreType.DMA,
        ],
    )
    def gather_kernel(seq_hbm, w_hbm, k_hbm, wg_hbm, kg_hbm,
                      idx_v, wrow_v, krow_v, gsem0, gsem1, wsem0, wsem1):
        wid = lax.axis_index("s") * _NC + lax.axis_index("c")
        base = wid * tok_per_w
        gsems = (gsem0, gsem1)
        wsems = (wsem0, wsem1)

        def wait_writeback(b):
            pltpu.make_async_copy(
                wrow_v.at[b], wg_hbm.at[pl.ds(base, _CHUNK)], wsems[b]).wait()
            pltpu.make_async_copy(
                krow_v.at[b], kg_hbm.at[pl.ds(base, _CHUNK)], wsems[b]).wait()

        @pl.loop(0, nround)
        def _round(g):
            @pl.when(g > 0)
            def _():
                wait_writeback(0)
                wait_writeback(1)

            copies = []
            for b in range(_NBUF):
                off = base + (g * _NBUF + b) * _CHUNK
                pltpu.sync_copy(seq_hbm.at[pl.ds(off, _CHUNK)], idx_v.at[b])
                cw = pltpu.async_copy(w_hbm.at[idx_v.at[b]], wrow_v.at[b],
                                      gsems[b])
                ck = pltpu.async_copy(k_hbm.at[idx_v.at[b]], krow_v.at[b],
                                      gsems[b])
                copies.append((cw, ck))
            for b in range(_NBUF):
                cw, ck = copies[b]
                cw.wait()
                ck.wait()
                off = base + (g * _NBUF + b) * _CHUNK
                pltpu.async_copy(wrow_v.at[b], wg_hbm.at[pl.ds(off, _CHUNK)],
                                 wsems[b])
                pltpu.async_copy(krow_v.at[b], kg_hbm.at[pl.ds(off, _CHUNK)],
                                 wsems[b])

        wait_writeback(0)
        wait_writeback(1)

    return gather_kernel(seq_grp, Wp, Kp)


def _tc_merge(Wg, Kg):
    bb = 16
    n3, n5, n7 = _L - 2, _L - 4, _L - 6

    def body(wg_ref, kg_ref, o3_ref, o5_ref, o7_ref):
        for b in range(bb):
            def prod(d, clo, n):
                w = wg_ref[b, pl.ds(clo + d, n), pl.ds(0, _EMB)]
                k = kg_ref[b, pl.ds(clo, n), pl.ds(_EMB * (3 + d), _EMB)]
                return w * k

            m = prod(-1, 1, n3)
            m = jnp.maximum(m, prod(0, 1, n3))
            m = jnp.maximum(m, prod(1, 1, n3))
            o3_ref[b] = m
            m = m[1:1 + n5]
            m = jnp.maximum(m, prod(-2, 2, n5))
            m = jnp.maximum(m, prod(2, 2, n5))
            o5_ref[b] = m
            m = m[1:1 + n7]
            m = jnp.maximum(m, prod(-3, 3, n7))
            m = jnp.maximum(m, prod(3, 3, n7))
            o7_ref[b] = m

    out = pl.pallas_call(
        body,
        grid=(_BG // bb,),
        in_specs=[
            pl.BlockSpec((bb, _L, _WPAD), lambda i: (i, 0, 0)),
            pl.BlockSpec((bb, _L, _KPAD), lambda i: (i, 0, 0)),
        ],
        out_specs=[
            pl.BlockSpec((bb, n3, _EMB), lambda i: (i, 0, 0)),
            pl.BlockSpec((bb, n5, _EMB), lambda i: (i, 0, 0)),
            pl.BlockSpec((bb, n7, _EMB), lambda i: (i, 0, 0)),
        ],
        out_shape=[
            jax.ShapeDtypeStruct((_BG, n3, _EMB), jnp.float32),
            jax.ShapeDtypeStruct((_BG, n5, _EMB), jnp.float32),
            jax.ShapeDtypeStruct((_BG, n7, _EMB), jnp.float32),
        ],
    )(Wg, Kg)
    return tuple(out)


@jax.jit
def kernel(seq, W, K):
    seq_flat = seq.astype(jnp.int32).reshape(-1)
    KT = jnp.transpose(K, (1, 2, 0)).reshape(_RMAX * _EMB, _VOCAB)
    WT = W.T
    tail0 = (_NBLK - 1) * _VB + _MAINW
    ktail = jnp.pad(KT[:, tail0:], ((0, 0), (0, 128 - (_VOCAB - tail0))))
    wtail = jnp.pad(WT[:, tail0:], ((0, 0), (0, 128 - (_VOCAB - tail0))))
    Kp, Wp = _tc_pack(KT, WT, ktail, wtail)

    parts = []
    for g in range(_NGRP):
        seq_grp = lax.slice(seq_flat, (g * _TOKG,), ((g + 1) * _TOKG,))
        Wg, Kg = _sc_gather(seq_grp, Wp, Kp)
        Wg = Wg.reshape(_BG, _L, _WPAD)
        Kg = Kg.reshape(_BG, _L, _KPAD)
        parts.append(_tc_merge(Wg, Kg))
    return tuple(jnp.concatenate([p[i] for p in parts], axis=0)
                 for i in range(3))


# speedup vs baseline: 1.0895x; 1.0895x over previous
import functools

import jax
import jax.numpy as jnp
from jax import lax
from jax.experimental import pallas as pl
from jax.experimental.pallas import tpu as pltpu
from jax.experimental.pallas import tpu_sc as plsc

_VOCAB = 100000
_EMB = 64
_RMAX = 7
_B = 1024
_L = 200
_NTOK = _B * _L

_WPAD = 128
_KPAD = 512

_NC, _NS = 2, 16
_NW = _NC * _NS
_NBUF = 2

_VB = 2048
_NBLK = (_VOCAB + _VB - 1) // _VB
_MAINW = (_VOCAB - (_NBLK - 1) * _VB) // 128 * 128
_VPAD = _NBLK * _VB

_NGRP = 1
_BG = _B // _NGRP
_TOKG = _BG * _L
_CHUNK = 80


def _tc_pack(KT, WT, ktail, wtail):
    def fetch(src_hbm, dst, sems, i, rows):
        b = lax.rem(i, 2)

        @pl.when(i < _NBLK - 1)
        def _():
            pltpu.make_async_copy(
                src_hbm.at[:, pl.ds(i * _VB, _VB)], dst.at[b], sems.at[b]
            ).start()

        @pl.when(i == _NBLK - 1)
        def _():
            pltpu.make_async_copy(
                src_hbm.at[:, pl.ds(i * _VB, _MAINW)],
                dst.at[b, :, pl.ds(0, _MAINW)], sems.at[b]
            ).start()

    def drain(src_hbm, dst, sems, i):
        b = lax.rem(i, 2)

        @pl.when(i < _NBLK - 1)
        def _():
            pltpu.make_async_copy(
                src_hbm.at[:, pl.ds(i * _VB, _VB)], dst.at[b], sems.at[b]
            ).wait()

        @pl.when(i == _NBLK - 1)
        def _():
            pltpu.make_async_copy(
                src_hbm.at[:, pl.ds(i * _VB, _MAINW)],
                dst.at[b, :, pl.ds(0, _MAINW)], sems.at[b]
            ).wait()

    def body(kt_hbm, wt_hbm, ktail_ref, wtail_ref, kp_ref, wp_ref,
             kt_v, wt_v, ksems, wsems):
        i = pl.program_id(0)
        b = lax.rem(i, 2)

        @pl.when(i == 0)
        def _():
            fetch(kt_hbm, kt_v, ksems, i, _VB)
            fetch(wt_hbm, wt_v, wsems, i, _VB)

        @pl.when(i < _NBLK - 1)
        def _():
            fetch(kt_hbm, kt_v, ksems, i + 1, _VB)
            fetch(wt_hbm, wt_v, wsems, i + 1, _VB)

        def kchunk(src_ref, lane0, s):
            cols = [src_ref[pl.ds(64 * j, 64), pl.ds(lane0, 128)].T
                    for j in range(_RMAX)]
            groups = [jnp.concatenate(cols[0:2], axis=1),
                      jnp.concatenate(cols[2:4], axis=1),
                      jnp.concatenate(cols[4:6], axis=1),
                      cols[6]]
            for g, val in enumerate(groups):
                kp_ref[pl.ds(128 * s, 128), pl.ds(128 * g, val.shape[1])] = val

        def wchunk(src_ref, lane0, s):
            wp_ref[pl.ds(128 * s, 128), pl.ds(0, 64)] = (
                src_ref[pl.ds(0, 64), pl.ds(lane0, 128)].T)

        drain(kt_hbm, kt_v, ksems, i)
        drain(wt_hbm, wt_v, wsems, i)

        @pl.when(i < _NBLK - 1)
        def _():
            for s in range(_VB // 128):
                kchunk(kt_v.at[b], 128 * s, s)
                wchunk(wt_v.at[b], 128 * s, s)

        @pl.when(i == _NBLK - 1)
        def _():
            for s in range(_MAINW // 128):
                kchunk(kt_v.at[b], 128 * s, s)
                wchunk(wt_v.at[b], 128 * s, s)
            kchunk(ktail_ref, 0, _MAINW // 128)
            wchunk(wtail_ref, 0, _MAINW // 128)

    return pl.pallas_call(
        body,
        grid=(_NBLK,),
        in_specs=[pl.BlockSpec(memory_space=pl.ANY),
                  pl.BlockSpec(memory_space=pl.ANY),
                  pl.BlockSpec((_RMAX * _EMB, 128), lambda i: (0, 0)),
                  pl.BlockSpec((_EMB, 128), lambda i: (0, 0))],
        out_specs=[pl.BlockSpec((_VB, _KPAD), lambda i: (i, 0)),
                   pl.BlockSpec((_VB, _WPAD), lambda i: (i, 0))],
        out_shape=[jax.ShapeDtypeStruct((_VPAD, _KPAD), jnp.float32),
                   jax.ShapeDtypeStruct((_VPAD, _WPAD), jnp.float32)],
        scratch_shapes=[
            pltpu.VMEM((2, _RMAX * _EMB, _VB), jnp.float32),
            pltpu.VMEM((2, _EMB, _VB), jnp.float32),
            pltpu.SemaphoreType.DMA((2,)),
            pltpu.SemaphoreType.DMA((2,)),
        ],
    )(KT, WT, ktail, wtail)


def _sc_gather(seq_grp, Wp, Kp):
    mesh = plsc.VectorSubcoreMesh(core_axis_name="c", subcore_axis_name="s")
    tok_per_w = _TOKG // _NW
    nround = tok_per_w // _CHUNK // _NBUF

    @functools.partial(
        pl.kernel,
        out_type=(
            jax.ShapeDtypeStruct((_TOKG, _WPAD), jnp.float32),
            jax.ShapeDtypeStruct((_TOKG, _KPAD), jnp.float32),
        ),
        mesh=mesh,
        scratch_types=[
            pltpu.VMEM((_NBUF, _CHUNK), jnp.int32),
            pltpu.VMEM((_NBUF, _CHUNK, _WPAD), jnp.float32),
            pltpu.VMEM((_NBUF, _CHUNK, _KPAD), jnp.float32),
            pltpu.SemaphoreType.DMA,
            pltpu.SemaphoreType.DMA,
            pltpu.SemaphoreType.DMA,
            pltpu.SemaphoreType.DMA,
        ],
    )
    def gather_kernel(seq_hbm, w_hbm, k_hbm, wg_hbm, kg_hbm,
                      idx_v, wrow_v, krow_v, gsem0, gsem1, wsem0, wsem1):
        wid = lax.axis_index("s") * _NC + lax.axis_index("c")
        base = wid * tok_per_w
        gsems = (gsem0, gsem1)
        wsems = (wsem0, wsem1)

        def wait_writeback(b):
            pltpu.make_async_copy(
                wrow_v.at[b], wg_hbm.at[pl.ds(base, _CHUNK)], wsems[b]).wait()
            pltpu.make_async_copy(
                krow_v.at[b], kg_hbm.at[pl.ds(base, _CHUNK)], wsems[b]).wait()

        @pl.loop(0, nround)
        def _round(g):
            @pl.when(g > 0)
            def _():
                wait_writeback(0)
                wait_writeback(1)

            copies = []
            for b in range(_NBUF):
                off = base + (g * _NBUF + b) * _CHUNK
                pltpu.sync_copy(seq_hbm.at[pl.ds(off, _CHUNK)], idx_v.at[b])
                cw = pltpu.async_copy(w_hbm.at[idx_v.at[b]], wrow_v.at[b],
                                      gsems[b])
                ck = pltpu.async_copy(k_hbm.at[idx_v.at[b]], krow_v.at[b],
                                      gsems[b])
                copies.append((cw, ck))
            for b in range(_NBUF):
                cw, ck = copies[b]
                cw.wait()
                ck.wait()
                off = base + (g * _NBUF + b) * _CHUNK
                pltpu.async_copy(wrow_v.at[b], wg_hbm.at[pl.ds(off, _CHUNK)],
                                 wsems[b])
                pltpu.async_copy(krow_v.at[b], kg_hbm.at[pl.ds(off, _CHUNK)],
                                 wsems[b])

        wait_writeback(0)
        wait_writeback(1)

    return gather_kernel(seq_grp, Wp, Kp)


def _tc_merge(Wg, Kg):
    bb = 16
    n3, n5, n7 = _L - 2, _L - 4, _L - 6

    def body(wg_ref, kg_ref, o3_ref, o5_ref, o7_ref):
        for b in range(bb):
            def prod(d, clo, n):
                w = wg_ref[b, pl.ds(clo + d, n), pl.ds(0, _EMB)]
                k = kg_ref[b, pl.ds(clo, n), pl.ds(_EMB * (3 + d), _EMB)]
                return w * k

            m = prod(-1, 1, n3)
            m = jnp.maximum(m, prod(0, 1, n3))
            m = jnp.maximum(m, prod(1, 1, n3))
            o3_ref[b] = m
            m = m[1:1 + n5]
            m = jnp.maximum(m, prod(-2, 2, n5))
            m = jnp.maximum(m, prod(2, 2, n5))
            o5_ref[b] = m
            m = m[1:1 + n7]
            m = jnp.maximum(m, prod(-3, 3, n7))
            m = jnp.maximum(m, prod(3, 3, n7))
            o7_ref[b] = m

    out = pl.pallas_call(
        body,
        grid=(_BG // bb,),
        in_specs=[
            pl.BlockSpec((bb, _L, _WPAD), lambda i: (i, 0, 0)),
            pl.BlockSpec((bb, _L, _KPAD), lambda i: (i, 0, 0)),
        ],
        out_specs=[
            pl.BlockSpec((bb, n3, _EMB), lambda i: (i, 0, 0)),
            pl.BlockSpec((bb, n5, _EMB), lambda i: (i, 0, 0)),
            pl.BlockSpec((bb, n7, _EMB), lambda i: (i, 0, 0)),
        ],
        out_shape=[
            jax.ShapeDtypeStruct((_BG, n3, _EMB), jnp.float32),
            jax.ShapeDtypeStruct((_BG, n5, _EMB), jnp.float32),
            jax.ShapeDtypeStruct((_BG, n7, _EMB), jnp.float32),
        ],
    )(Wg, Kg)
    return tuple(out)


@jax.jit
def kernel(seq, W, K):
    seq_flat = seq.astype(jnp.int32).reshape(-1)
    KT = jnp.transpose(K, (1, 2, 0)).reshape(_RMAX * _EMB, _VOCAB)
    WT = W.T
    tail0 = (_NBLK - 1) * _VB + _MAINW
    ktail = jnp.pad(KT[:, tail0:], ((0, 0), (0, 128 - (_VOCAB - tail0))))
    wtail = jnp.pad(WT[:, tail0:], ((0, 0), (0, 128 - (_VOCAB - tail0))))
    Kp, Wp = _tc_pack(KT, WT, ktail, wtail)

    parts = []
    for g in range(_NGRP):
        seq_grp = lax.slice(seq_flat, (g * _TOKG,), ((g + 1) * _TOKG,))
        Wg, Kg = _sc_gather(seq_grp, Wp, Kp)
        Wg = Wg.reshape(_BG, _L, _WPAD)
        Kg = Kg.reshape(_BG, _L, _KPAD)
        parts.append(_tc_merge(Wg, Kg))
    return tuple(jnp.concatenate([p[i] for p in parts], axis=0)
                 for i in range(3))
